# baseline (speedup 1.0000x reference)
import functools

import jax
from jax import lax
import jax.numpy as jnp
from jax.experimental import pallas as pl
from jax.experimental.pallas import tpu as pltpu
from jax.experimental.pallas import tpu_sc as plsc

_K = 16
_SCALE = 2
_NC, _NS = 2, 16


def _topk_body(spc_ref, spr_ref, sqc_ref, sqr_ref, score_ref, idx_ref, esk_ref,
               *, n_cols):
    xi = spc_ref[:, 0:1].astype(jnp.float32)
    yi = spc_ref[:, 1:2].astype(jnp.float32)
    xr = spr_ref[0:1, :].astype(jnp.float32)
    yr = spr_ref[1:2, :].astype(jnp.float32)
    t1 = xi * xr
    t2 = yi * yr
    dot = t1 + t2
    d2 = (sqc_ref[...] + sqr_ref[...]) - 2.0 * dot
    d2 = jnp.maximum(d2, 0.0)

    m_rows = d2.shape[0]
    big = jnp.int32(n_cols)
    inf = jnp.float32(jnp.inf)
    nlane = 128
    nq = n_cols // nlane

    def ce_static(a, b):
        p = a[0] <= b[0]
        lo = (jnp.where(p, a[0], b[0]), jnp.where(p, a[1], b[1]))
        hi = (jnp.where(p, b[0], a[0]), jnp.where(p, b[1], a[1]))
        return lo, hi

    def ce_lex(a, b):
        p = (a[0] < b[0]) | ((a[0] == b[0]) & (a[1] < b[1]))
        lo = (jnp.where(p, a[0], b[0]), jnp.where(p, a[1], b[1]))
        hi = (jnp.where(p, b[0], a[0]), jnp.where(p, b[1], a[1]))
        return lo, hi

    iota_l = jax.lax.broadcasted_iota(jnp.int32, (m_rows, nlane), 1)
    items = [[(d2[:, q * nlane:(q + 1) * nlane], iota_l + q * nlane)]
             for q in range(nq)]
    items = [list(ce_static(a[0], b[0]))
             for a, b in zip(items[0::2], items[1::2])]

    def merge_keepall_22(a, b):
        l0, h0 = ce_static(a[0], b[1])
        l1, h1 = ce_static(a[1], b[0])
        l0, l1 = ce_lex(l0, l1)
        h0, h1 = ce_lex(h0, h1)
        return [l0, l1, h0, h1]

    items = [merge_keepall_22(a, b)
             for a, b in zip(items[0::2], items[1::2])]

    def merge_keep4(a, b):
        lo = [ce_static(a[i], b[3 - i])[0] for i in range(4)]
        lo[0], lo[2] = ce_lex(lo[0], lo[2])
        lo[1], lo[3] = ce_lex(lo[1], lo[3])
        lo[0], lo[1] = ce_lex(lo[0], lo[1])
        lo[2], lo[3] = ce_lex(lo[2], lo[3])
        return lo

    while len(items) > 1:
        items = [merge_keep4(a, b)
                 for a, b in zip(items[0::2], items[1::2])]
    lv = [s[0] for s in items[0]]
    lj = [s[1] for s in items[0]]

    cnt = jnp.zeros((m_rows, nlane), jnp.int32)
    outs_v, outs_j = [], []
    for _ in range(_K):
        mn = jnp.min(lv[0], axis=1, keepdims=True)
        eq = lv[0] == mn
        jm = jnp.min(jnp.where(eq, lj[0], big), axis=1, keepdims=True)
        pop = eq & (lj[0] == jm)
        outs_v.append(mn)
        outs_j.append(jm)
        for dd in range(3):
            lv[dd] = jnp.where(pop, lv[dd + 1], lv[dd])
            lj[dd] = jnp.where(pop, lj[dd + 1], lj[dd])
        lv[3] = jnp.where(pop, inf, lv[3])
        lj[3] = jnp.where(pop, big, lj[3])
        cnt = cnt + pop.astype(jnp.int32)
    sc_fast = jnp.concatenate(outs_v, axis=1)
    ix_fast = jnp.concatenate(outs_j, axis=1)
    flag = jnp.any(cnt >= 4)

    def _slow(_):
        iota = jax.lax.broadcasted_iota(jnp.int32, (m_rows, n_cols), 1)
        vals = d2
        scores = []
        idxs = []
        for _ in range(_K):
            mn = jnp.min(vals, axis=1, keepdims=True)
            ji = jnp.min(jnp.where(vals == mn, iota, big), axis=1,
                         keepdims=True)
            scores.append(mn)
            idxs.append(ji)
            vals = jnp.where(iota == ji, inf, vals)
        return jnp.concatenate(scores, axis=1), jnp.concatenate(idxs, axis=1)

    def _fast(_):
        return sc_fast, ix_fast

    sc, ix = jax.lax.cond(flag, _slow, _fast, None)
    d = jnp.sqrt(sc + 1e-12)
    score_ref[...] = d
    idx_ref[...] = ix
    esk_ref[...] = jnp.exp(d * (-0.1))


def _sc_gather_rows(idx_flat, xe, n_rows_out, chunk=512):
    e = xe.shape[1]
    nw = _NC * _NS
    bpw = n_rows_out // nw
    mesh = plsc.VectorSubcoreMesh(core_axis_name="c", subcore_axis_name="s")

    @functools.partial(
        pl.kernel, mesh=mesh,
        out_type=jax.ShapeDtypeStruct((n_rows_out, e), jnp.float32),
        scratch_types=[
            pltpu.VMEM((chunk,), jnp.int32),
            pltpu.VMEM((chunk, e), jnp.float32),
            pltpu.SemaphoreType.DMA,
        ])
    def _gather(idx_hbm, xe_hbm, out_hbm, idx_v, rows_v, sem):
        wid = lax.axis_index("s") * _NC + lax.axis_index("c")
        base = wid * bpw

        @pl.loop(0, bpw // chunk)
        def _(ci):
            off = base + ci * chunk
            pltpu.sync_copy(idx_hbm.at[pl.ds(off, chunk)], idx_v)
            pltpu.async_copy(xe_hbm.at[idx_v], rows_v, sem).wait()
            pltpu.sync_copy(rows_v, out_hbm.at[pl.ds(off, chunk)])

    return _gather(idx_flat, xe)


def _diff_body(g_ref, yet_ref, dp_ref):
    g = g_ref[0]
    e = g.shape[1]
    ident = (jax.lax.broadcasted_iota(jnp.int32, (e, e), 0) ==
             jax.lax.broadcasted_iota(jnp.int32, (e, e), 1)).astype(
                 jnp.float32)
    gt = jax.lax.dot_general(
        ident, g,
        dimension_numbers=(((1,), (1,)), ((), ())),
        precision=jax.lax.Precision.HIGHEST,
        preferred_element_type=jnp.float32)
    d = jnp.abs(yet_ref[...] - gt)
    dp_ref[0, :, 0, :] = d
    dp_ref[0, :, 1, :] = d


def kernel(xe_patch, ye_patch, spatial):
    n, e = xe_patch.shape
    k = _K

    sq = jnp.sum(spatial * spatial, axis=1)
    sq_col = sq.reshape(n, 1)
    sq_row = sq.reshape(1, n)
    spb = spatial.astype(jnp.bfloat16)
    sp_t = spb.T
    yet = ye_patch.T

    m1 = 128
    grid1 = n // m1
    score, idx, esk = pl.pallas_call(
        functools.partial(_topk_body, n_cols=n),
        grid=(grid1,),
        in_specs=[
            pl.BlockSpec((m1, 2), lambda i: (i, 0)),
            pl.BlockSpec((2, n), lambda i: (0, 0)),
            pl.BlockSpec((m1, 1), lambda i: (i, 0)),
            pl.BlockSpec((1, n), lambda i: (0, 0)),
        ],
        out_specs=[
            pl.BlockSpec((m1, k), lambda i: (i, 0)),
            pl.BlockSpec((m1, k), lambda i: (i, 0)),
            pl.BlockSpec((m1, k), lambda i: (i, 0)),
        ],
        out_shape=[
            jax.ShapeDtypeStruct((n, k), jnp.float32),
            jax.ShapeDtypeStruct((n, k), jnp.int32),
            jax.ShapeDtypeStruct((n, k), jnp.float32),
        ],
        compiler_params=pltpu.CompilerParams(
            dimension_semantics=("parallel",)),
    )(spb, sp_t, sq_col, sq_row)

    idx_flat = idx.T.reshape(k * n)
    g = _sc_gather_rows(idx_flat, xe_patch, k * n)
    g3 = g.reshape(k, n, e)

    mb = 512
    grid2 = n // mb
    dp4 = pl.pallas_call(
        _diff_body,
        grid=(k, grid2),
        in_specs=[
            pl.BlockSpec((1, mb, e), lambda i, j: (i, j, 0)),
            pl.BlockSpec((e, mb), lambda i, j: (0, j)),
        ],
        out_specs=pl.BlockSpec((1, e, _SCALE, mb), lambda i, j: (i, 0, 0, j)),
        out_shape=jax.ShapeDtypeStruct((k, e, _SCALE, n), jnp.float32),
        compiler_params=pltpu.CompilerParams(
            dimension_semantics=("parallel", "parallel")),
    )(g3, yet)

    sk = jnp.broadcast_to(esk.T[:, None, :], (k, _SCALE, n)).reshape(
        1, k, _SCALE * n)
    dp = dp4.reshape(1, k * e, _SCALE * n)
    return (sk, idx[None], dp)


# speedup vs baseline: 1.0374x; 1.0374x over previous
import functools

import jax
from jax import lax
import jax.numpy as jnp
from jax.experimental import pallas as pl
from jax.experimental.pallas import tpu as pltpu
from jax.experimental.pallas import tpu_sc as plsc

_K = 16
_SCALE = 2
_NC, _NS = 2, 16


def _topk_body(spc_ref, spr_ref, sqc_ref, sqr_ref, score_ref, idx_ref, esk_ref,
               *, n_cols):
    xi = spc_ref[:, 0:1].astype(jnp.float32)
    yi = spc_ref[:, 1:2].astype(jnp.float32)
    xr = spr_ref[0:1, :].astype(jnp.float32)
    yr = spr_ref[1:2, :].astype(jnp.float32)
    t1 = xi * xr
    t2 = yi * yr
    dot = t1 + t2
    d2 = (sqc_ref[...] + sqr_ref[...]) - 2.0 * dot
    d2 = jnp.maximum(d2, 0.0)

    m_rows = d2.shape[0]
    big = jnp.int32(n_cols)
    inf = jnp.float32(jnp.inf)
    nlane = 128
    nq = n_cols // nlane

    def ce_static(a, b):
        p = a[0] <= b[0]
        lo = (jnp.where(p, a[0], b[0]), jnp.where(p, a[1], b[1]))
        hi = (jnp.where(p, b[0], a[0]), jnp.where(p, b[1], a[1]))
        return lo, hi

    def ce_lex(a, b):
        p = (a[0] < b[0]) | ((a[0] == b[0]) & (a[1] < b[1]))
        lo = (jnp.minimum(a[0], b[0]), jnp.where(p, a[1], b[1]))
        hi = (jnp.maximum(a[0], b[0]), jnp.where(p, b[1], a[1]))
        return lo, hi

    iota_l = jax.lax.broadcasted_iota(jnp.int32, (m_rows, nlane), 1)
    items = [[(d2[:, q * nlane:(q + 1) * nlane], iota_l + q * nlane)]
             for q in range(nq)]
    items = [list(ce_static(a[0], b[0]))
             for a, b in zip(items[0::2], items[1::2])]

    def merge_keepall_22(a, b):
        l0, h0 = ce_static(a[0], b[1])
        l1, h1 = ce_static(a[1], b[0])
        l0, l1 = ce_lex(l0, l1)
        h0, h1 = ce_lex(h0, h1)
        return [l0, l1, h0, h1]

    items = [merge_keepall_22(a, b)
             for a, b in zip(items[0::2], items[1::2])]

    def merge_keep4(a, b):
        lo = [ce_static(a[i], b[3 - i])[0] for i in range(4)]
        lo[0], lo[2] = ce_lex(lo[0], lo[2])
        lo[1], lo[3] = ce_lex(lo[1], lo[3])
        lo[0], lo[1] = ce_lex(lo[0], lo[1])
        lo[2], lo[3] = ce_lex(lo[2], lo[3])
        return lo

    while len(items) > 1:
        items = [merge_keep4(a, b)
                 for a, b in zip(items[0::2], items[1::2])]
    lv = [s[0] for s in items[0]]
    lj = [s[1] for s in items[0]]

    cnt = jnp.zeros((m_rows, nlane), jnp.int32)
    outs_v, outs_j = [], []
    for _ in range(_K):
        mn = jnp.min(lv[0], axis=1, keepdims=True)
        eq = lv[0] == mn
        jm = jnp.min(jnp.where(eq, lj[0], big), axis=1, keepdims=True)
        pop = eq & (lj[0] == jm)
        outs_v.append(mn)
        outs_j.append(jm)
        for dd in range(3):
            lv[dd] = jnp.where(pop, lv[dd + 1], lv[dd])
            lj[dd] = jnp.where(pop, lj[dd + 1], lj[dd])
        lv[3] = jnp.where(pop, inf, lv[3])
        lj[3] = jnp.where(pop, big, lj[3])
        cnt = cnt + pop.astype(jnp.int32)
    sc_fast = jnp.concatenate(outs_v, axis=1)
    ix_fast = jnp.concatenate(outs_j, axis=1)
    flag = jnp.any(cnt >= 4)

    def _slow(_):
        iota = jax.lax.broadcasted_iota(jnp.int32, (m_rows, n_cols), 1)
        vals = d2
        scores = []
        idxs = []
        for _ in range(_K):
            mn = jnp.min(vals, axis=1, keepdims=True)
            ji = jnp.min(jnp.where(vals == mn, iota, big), axis=1,
                         keepdims=True)
            scores.append(mn)
            idxs.append(ji)
            vals = jnp.where(iota == ji, inf, vals)
        return jnp.concatenate(scores, axis=1), jnp.concatenate(idxs, axis=1)

    def _fast(_):
        return sc_fast, ix_fast

    sc, ix = jax.lax.cond(flag, _slow, _fast, None)
    d = jnp.sqrt(sc + 1e-12)
    score_ref[...] = d
    idx_ref[...] = ix
    esk_ref[...] = jnp.exp(d * (-0.1))


def _sc_gather_rows(idx_flat, xe, n_rows_out, chunk=512):
    e = xe.shape[1]
    nw = _NC * _NS
    bpw = n_rows_out // nw
    mesh = plsc.VectorSubcoreMesh(core_axis_name="c", subcore_axis_name="s")

    @functools.partial(
        pl.kernel, mesh=mesh,
        out_type=jax.ShapeDtypeStruct((n_rows_out, e), jnp.float32),
        scratch_types=[
            pltpu.VMEM((chunk,), jnp.int32),
            pltpu.VMEM((chunk, e), jnp.float32),
            pltpu.SemaphoreType.DMA,
        ])
    def _gather(idx_hbm, xe_hbm, out_hbm, idx_v, rows_v, sem):
        wid = lax.axis_index("s") * _NC + lax.axis_index("c")
        base = wid * bpw

        @pl.loop(0, bpw // chunk)
        def _(ci):
            off = base + ci * chunk
            pltpu.sync_copy(idx_hbm.at[pl.ds(off, chunk)], idx_v)
            pltpu.async_copy(xe_hbm.at[idx_v], rows_v, sem).wait()
            pltpu.sync_copy(rows_v, out_hbm.at[pl.ds(off, chunk)])

    return _gather(idx_flat, xe)


def _diff_body(g_ref, yet_ref, dp_ref):
    g = g_ref[0]
    gt = jnp.transpose(g, (1, 0))
    d = jnp.abs(yet_ref[...] - gt)
    dp_ref[0, :, 0, :] = d
    dp_ref[0, :, 1, :] = d


def kernel(xe_patch, ye_patch, spatial):
    n, e = xe_patch.shape
    k = _K

    sq = jnp.sum(spatial * spatial, axis=1)
    sq_col = sq.reshape(n, 1)
    sq_row = sq.reshape(1, n)
    spb = spatial.astype(jnp.bfloat16)
    sp_t = spb.T
    yet = ye_patch.T

    m1 = 128
    grid1 = n // m1
    score, idx, esk = pl.pallas_call(
        functools.partial(_topk_body, n_cols=n),
        grid=(grid1,),
        in_specs=[
            pl.BlockSpec((m1, 2), lambda i: (i, 0)),
            pl.BlockSpec((2, n), lambda i: (0, 0)),
            pl.BlockSpec((m1, 1), lambda i: (i, 0)),
            pl.BlockSpec((1, n), lambda i: (0, 0)),
        ],
        out_specs=[
            pl.BlockSpec((m1, k), lambda i: (i, 0)),
            pl.BlockSpec((m1, k), lambda i: (i, 0)),
            pl.BlockSpec((m1, k), lambda i: (i, 0)),
        ],
        out_shape=[
            jax.ShapeDtypeStruct((n, k), jnp.float32),
            jax.ShapeDtypeStruct((n, k), jnp.int32),
            jax.ShapeDtypeStruct((n, k), jnp.float32),
        ],
        compiler_params=pltpu.CompilerParams(
            dimension_semantics=("parallel",)),
    )(spb, sp_t, sq_col, sq_row)

    idx_flat = idx.T.reshape(k * n)
    g = _sc_gather_rows(idx_flat, xe_patch, k * n)
    g3 = g.reshape(k, n, e)

    mb = 1024
    grid2 = n // mb
    dp4 = pl.pallas_call(
        _diff_body,
        grid=(grid2, k),
        in_specs=[
            pl.BlockSpec((1, mb, e), lambda j, i: (i, j, 0)),
            pl.BlockSpec((e, mb), lambda j, i: (0, j)),
        ],
        out_specs=pl.BlockSpec((1, e, _SCALE, mb), lambda j, i: (i, 0, 0, j)),
        out_shape=jax.ShapeDtypeStruct((k, e, _SCALE, n), jnp.float32),
        compiler_params=pltpu.CompilerParams(
            dimension_semantics=("parallel", "parallel")),
    )(g3, yet)

    sk = jnp.broadcast_to(esk.T[:, None, :], (k, _SCALE, n)).reshape(
        1, k, _SCALE * n)
    dp = dp4.reshape(1, k * e, _SCALE * n)
    return (sk, idx[None], dp)
